# baseline (speedup 1.0000x reference)
import functools

import jax
import jax.numpy as jnp
from jax.experimental import pallas as pl
from jax.experimental.pallas import tpu as pltpu

TOP_K = 2
N_EXPERTS = 8
LANES = 128
NEG = -1e30


def _gate_body(x_ref, w_ref, idx_ref, wgt_ref):
    x = x_ref[...]
    w = w_ref[...]
    logits = jnp.dot(x, w, preferred_element_type=jnp.float32)
    lane = jax.lax.broadcasted_iota(jnp.int32, logits.shape, 1)
    l1 = jnp.where(lane < N_EXPERTS, logits, NEG)
    m1 = jnp.max(l1, axis=1, keepdims=True)
    i1 = jnp.min(jnp.where(l1 == m1, lane, LANES), axis=1, keepdims=True)
    l2 = jnp.where(lane == i1, NEG, l1)
    m2 = jnp.max(l2, axis=1, keepdims=True)
    i2 = jnp.min(jnp.where(l2 == m2, lane, LANES), axis=1, keepdims=True)
    e = jnp.exp(m2 - m1)
    w1 = 1.0 / (1.0 + e)
    w2 = e * w1
    idx_ref[...] = jnp.concatenate([i1, i2], axis=1)
    wgt_ref[...] = jnp.concatenate([w1, w2], axis=1)


@functools.partial(jax.jit, static_argnames=("block_m",))
def _gate(x, wp, block_m):
    tokens, h = x.shape
    grid = tokens // block_m
    return pl.pallas_call(
        _gate_body,
        grid=(grid,),
        in_specs=[
            pl.BlockSpec((block_m, h), lambda i: (i, 0)),
            pl.BlockSpec((h, LANES), lambda i: (0, 0)),
        ],
        out_specs=[
            pl.BlockSpec((block_m, TOP_K), lambda i: (i, 0)),
            pl.BlockSpec((block_m, TOP_K), lambda i: (i, 0)),
        ],
        out_shape=[
            jax.ShapeDtypeStruct((tokens, TOP_K), jnp.int32),
            jax.ShapeDtypeStruct((tokens, TOP_K), jnp.float32),
        ],
        compiler_params=pltpu.CompilerParams(
            dimension_semantics=("arbitrary",),
        ),
    )(x, wp)


def kernel(hidden_states, weight):
    bsz, seq_len, h = hidden_states.shape
    x = hidden_states.reshape(bsz * seq_len, h)
    wp = jnp.zeros((h, LANES), jnp.float32).at[:, :N_EXPERTS].set(weight.T)
    topk_idx, topk_weight = _gate(x, wp, 2048)
    aux_loss = jnp.zeros((), jnp.float32)
    return topk_idx, topk_weight, aux_loss


# speedup vs baseline: 2.0040x; 2.0040x over previous
import functools

import jax
import jax.numpy as jnp
from jax.experimental import pallas as pl
from jax.experimental.pallas import tpu as pltpu

TOP_K = 2
N_EXPERTS = 8
NEG = -1e30


def _gate_body(x_ref, w_ref, idx_ref, wgt_ref):
    x = x_ref[...]
    w = w_ref[...]
    logits = jnp.dot(x, w, preferred_element_type=jnp.float32)
    lt = logits.T
    row = jax.lax.broadcasted_iota(jnp.int32, lt.shape, 0)
    m1 = jnp.max(lt, axis=0, keepdims=True)
    i1 = jnp.min(jnp.where(lt == m1, row, N_EXPERTS), axis=0, keepdims=True)
    l2 = jnp.where(row == i1, NEG, lt)
    m2 = jnp.max(l2, axis=0, keepdims=True)
    i2 = jnp.min(jnp.where(l2 == m2, row, N_EXPERTS), axis=0, keepdims=True)
    e = jnp.exp(m2 - m1)
    w1 = 1.0 / (1.0 + e)
    w2 = e * w1
    idx_ref[...] = jnp.concatenate([i1, i2], axis=0)
    wgt_ref[...] = jnp.concatenate([w1, w2], axis=0)


@functools.partial(jax.jit, static_argnames=("block_m",))
def _gate(x, wp, block_m):
    tokens, h = x.shape
    grid = tokens // block_m
    return pl.pallas_call(
        _gate_body,
        grid=(grid,),
        in_specs=[
            pl.BlockSpec((block_m, h), lambda i: (i, 0)),
            pl.BlockSpec((h, N_EXPERTS), lambda i: (0, 0)),
        ],
        out_specs=[
            pl.BlockSpec((TOP_K, block_m), lambda i: (0, i)),
            pl.BlockSpec((TOP_K, block_m), lambda i: (0, i)),
        ],
        out_shape=[
            jax.ShapeDtypeStruct((TOP_K, tokens), jnp.int32),
            jax.ShapeDtypeStruct((TOP_K, tokens), jnp.float32),
        ],
        compiler_params=pltpu.CompilerParams(
            dimension_semantics=("arbitrary",),
        ),
    )(x, wp)


def kernel(hidden_states, weight):
    bsz, seq_len, h = hidden_states.shape
    x = hidden_states.reshape(bsz * seq_len, h)
    idx_t, wgt_t = _gate(x, weight.T, 2048)
    topk_idx = idx_t.T
    topk_weight = wgt_t.T
    aux_loss = jnp.zeros((), jnp.float32)
    return topk_idx, topk_weight, aux_loss
